# Initial kernel scaffold; baseline (speedup 1.0000x reference)
import functools

import jax
import jax.numpy as jnp
from jax import lax
from jax.experimental import pallas as pl
from jax.experimental.pallas import tpu as pltpu
from jax.experimental.pallas import tpu_sc as plsc

B = 16384
F = 26
K = 16
NC = 2
NS = 16
NW = NC * NS
SPW = B // NW
CHUNK = 64
NCHUNK = SPW // CHUNK
ROWS = CHUNK * F


def _fm_body(xf_hbm, w_hbm, v_hbm, out_hbm, lin_hbm, int_hbm,
             idx_v, rows_v, w_v, lin_v, int_v, o_v, sem_v, sem_w):
    wid = lax.axis_index("s") * NC + lax.axis_index("c")
    lanes = lax.iota(jnp.int32, (16,), 0)
    zeros16 = jnp.zeros((16,), jnp.int32)

    def chunk_body(c, carry):
        sbase = wid * SPW + c * CHUNK
        ibase = sbase * F
        pltpu.sync_copy(xf_hbm.at[pl.ds(ibase, ROWS)], idx_v)
        cp_v = pltpu.async_copy(v_hbm.at[idx_v], rows_v, sem_v)
        cp_w = pltpu.async_copy(w_hbm.at[idx_v], w_v, sem_w)
        cp_v.wait()
        cp_w.wait()

        def sample_body(i, carry2):
            base = i * F
            r0 = rows_v[base]
            r1 = rows_v[base + 1]
            r2 = rows_v[base + 2]
            r3 = rows_v[base + 3]
            s0, s1, s2, s3 = r0, r1, r2, r3
            q0, q1, q2, q3 = r0 * r0, r1 * r1, r2 * r2, r3 * r3
            for f in range(4, F, 4):
                a0 = rows_v[base + f]
                a1 = rows_v[base + f + 1]
                s0 = s0 + a0
                q0 = q0 + a0 * a0
                s1 = s1 + a1
                q1 = q1 + a1 * a1
                if f + 2 < F:
                    a2 = rows_v[base + f + 2]
                    a3 = rows_v[base + f + 3]
                    s2 = s2 + a2
                    q2 = q2 + a2 * a2
                    s3 = s3 + a3
                    q3 = q3 + a3 * a3
            s = (s0 + s1) + (s2 + s3)
            q = (q0 + q1) + (q2 + q3)
            t = s * s - q
            int_v[i] = 0.5 * jnp.sum(t)
            return carry2

        lax.fori_loop(0, CHUNK, sample_body, 0)

        for g in range(CHUNK // 16):
            rowbase = (g * 16 + lanes) * F
            acc0 = plsc.load_gather(w_v, [rowbase, zeros16])
            acc1 = plsc.load_gather(w_v, [rowbase + 1, zeros16])
            for f in range(2, F, 2):
                acc0 = acc0 + plsc.load_gather(w_v, [rowbase + f, zeros16])
                acc1 = acc1 + plsc.load_gather(w_v, [rowbase + f + 1, zeros16])
            lin_v[pl.ds(g * 16, 16)] = acc0 + acc1

        for g in range(CHUNK // 16):
            z = lin_v[pl.ds(g * 16, 16)] + int_v[pl.ds(g * 16, 16)]
            o_v[pl.ds(g * 16, 16)] = 1.0 / (1.0 + jnp.exp(-z))

        pltpu.sync_copy(lin_v, lin_hbm.at[pl.ds(sbase, CHUNK)])
        pltpu.sync_copy(int_v, int_hbm.at[pl.ds(sbase, CHUNK)])
        pltpu.sync_copy(o_v, out_hbm.at[pl.ds(sbase, CHUNK)])
        return carry

    lax.fori_loop(0, NCHUNK, chunk_body, 0)


_fm_kernel = pl.kernel(
    _fm_body,
    out_type=(
        jax.ShapeDtypeStruct((B,), jnp.float32),
        jax.ShapeDtypeStruct((B,), jnp.float32),
        jax.ShapeDtypeStruct((B,), jnp.float32),
    ),
    mesh=plsc.VectorSubcoreMesh(core_axis_name="c", subcore_axis_name="s"),
    scratch_types=(
        pltpu.VMEM((ROWS,), jnp.int32),
        pltpu.VMEM((ROWS, K), jnp.float32),
        pltpu.VMEM((ROWS, 1), jnp.float32),
        pltpu.VMEM((CHUNK,), jnp.float32),
        pltpu.VMEM((CHUNK,), jnp.float32),
        pltpu.VMEM((CHUNK,), jnp.float32),
        pltpu.SemaphoreType.DMA,
        pltpu.SemaphoreType.DMA,
    ),
)


@jax.jit
def kernel(x, w_table, V_table):
    xf = x.reshape(-1)
    out, lin, inter = _fm_kernel(xf, w_table, V_table)
    return (out.reshape(B, 1), lin.reshape(B, 1), inter.reshape(B, 1))


# speedup vs baseline: 1.1666x; 1.1666x over previous
import jax
import jax.numpy as jnp
from jax import lax
from jax.experimental import pallas as pl
from jax.experimental.pallas import tpu as pltpu
from jax.experimental.pallas import tpu_sc as plsc

B = 16384
F = 26
K = 16
NC = 2
NS = 16
NW = NC * NS
SPW = B // NW
CHUNK = 64
NCHUNK = SPW // CHUNK
ROWS = CHUNK * F
NG = CHUNK // 16


def _fm_body(xf_hbm, w_hbm, v_hbm, out_hbm, lin_hbm, int_hbm,
             idx_v, rows_v, w_v, lin_v, int_v, o_v, sem_v, sem_w):
    wid = lax.axis_index("s") * NC + lax.axis_index("c")
    lanes = lax.iota(jnp.int32, 16)
    zeros16 = jnp.zeros((16,), jnp.int32)

    def chunk_body(c, carry):
        sbase = wid * SPW + c * CHUNK
        ibase = sbase * F
        pltpu.sync_copy(xf_hbm.at[pl.ds(ibase, ROWS)], idx_v)
        cp_v = pltpu.async_copy(v_hbm.at[idx_v], rows_v, sem_v)
        cp_w = pltpu.async_copy(w_hbm.at[idx_v], w_v, sem_w)
        cp_v.wait()
        cp_w.wait()

        def group_body(g, carry2):
            rowbase = (g * 16 + lanes) * F

            lin0 = plsc.load_gather(w_v, [rowbase])
            lin1 = plsc.load_gather(w_v, [rowbase + 1])
            for f in range(2, F, 2):
                lin0 = lin0 + plsc.load_gather(w_v, [rowbase + f])
                lin1 = lin1 + plsc.load_gather(w_v, [rowbase + f + 1])
            lin = lin0 + lin1

            acc_sq = jnp.zeros((16,), jnp.float32)
            acc_q0 = jnp.zeros((16,), jnp.float32)
            acc_q1 = jnp.zeros((16,), jnp.float32)
            for k in range(K):
                kvec = jnp.full((16,), k, jnp.int32)
                v0 = plsc.load_gather(rows_v, [rowbase, kvec])
                v1 = plsc.load_gather(rows_v, [rowbase + 1, kvec])
                s0 = v0
                s1 = v1
                q0 = v0 * v0
                q1 = v1 * v1
                for f in range(2, F, 2):
                    a0 = plsc.load_gather(rows_v, [rowbase + f, kvec])
                    a1 = plsc.load_gather(rows_v, [rowbase + f + 1, kvec])
                    s0 = s0 + a0
                    q0 = q0 + a0 * a0
                    s1 = s1 + a1
                    q1 = q1 + a1 * a1
                s_k = s0 + s1
                acc_sq = acc_sq + s_k * s_k
                acc_q0 = acc_q0 + q0
                acc_q1 = acc_q1 + q1
            inter = 0.5 * (acc_sq - (acc_q0 + acc_q1))

            z = lin + inter
            out = 1.0 / (1.0 + jnp.exp(-z))
            lin_v[pl.ds(g * 16, 16)] = lin
            int_v[pl.ds(g * 16, 16)] = inter
            o_v[pl.ds(g * 16, 16)] = out
            return carry2

        lax.fori_loop(0, NG, group_body, 0)

        pltpu.sync_copy(lin_v, lin_hbm.at[pl.ds(sbase, CHUNK)])
        pltpu.sync_copy(int_v, int_hbm.at[pl.ds(sbase, CHUNK)])
        pltpu.sync_copy(o_v, out_hbm.at[pl.ds(sbase, CHUNK)])
        return carry

    lax.fori_loop(0, NCHUNK, chunk_body, 0)


_fm_kernel = pl.kernel(
    _fm_body,
    out_type=(
        jax.ShapeDtypeStruct((B,), jnp.float32),
        jax.ShapeDtypeStruct((B,), jnp.float32),
        jax.ShapeDtypeStruct((B,), jnp.float32),
    ),
    mesh=plsc.VectorSubcoreMesh(core_axis_name="c", subcore_axis_name="s"),
    compiler_params=pltpu.CompilerParams(
        needs_layout_passes=False, use_tc_tiling_on_sc=False),
    scratch_types=(
        pltpu.VMEM((ROWS,), jnp.int32),
        pltpu.VMEM((ROWS, K), jnp.float32),
        pltpu.VMEM((ROWS,), jnp.float32),
        pltpu.VMEM((CHUNK,), jnp.float32),
        pltpu.VMEM((CHUNK,), jnp.float32),
        pltpu.VMEM((CHUNK,), jnp.float32),
        pltpu.SemaphoreType.DMA,
        pltpu.SemaphoreType.DMA,
    ),
)


@jax.jit
def kernel(x, w_table, V_table):
    xf = x.reshape(-1)
    out, lin, inter = _fm_kernel(xf, w_table.reshape(-1), V_table)
    return (out.reshape(B, 1), lin.reshape(B, 1), inter.reshape(B, 1))


# speedup vs baseline: 1.2103x; 1.0374x over previous
import jax
import jax.numpy as jnp
from jax import lax
from jax.experimental import pallas as pl
from jax.experimental.pallas import tpu as pltpu
from jax.experimental.pallas import tpu_sc as plsc

B = 16384
F = 26
K = 16
NC = 2
NS = 16
NW = NC * NS
SPW = B // NW
CHUNK = 64
NCHUNK = SPW // CHUNK
ROWS = CHUNK * F
NG = CHUNK // 16


def _fm_body(xf_hbm, w_hbm, v_hbm, out_hbm, lin_hbm, int_hbm,
             idx_v0, idx_v1, rows_v0, rows_v1, w_v0, w_v1,
             lin_v, int_v, o_v, sem_v0, sem_v1, sem_w0, sem_w1):
    wid = lax.axis_index("s") * NC + lax.axis_index("c")
    lanes = lax.iota(jnp.int32, 16)
    idx_b = (idx_v0, idx_v1)
    rows_b = (rows_v0, rows_v1)
    w_b = (w_v0, w_v1)
    sem_vb = (sem_v0, sem_v1)
    sem_wb = (sem_w0, sem_w1)

    def start_chunk(c):
        p = c & 1
        sbase = wid * SPW + c * CHUNK
        pltpu.sync_copy(xf_hbm.at[pl.ds(sbase * F, ROWS)], idx_b[p])
        cp_v = pltpu.async_copy(v_hbm.at[idx_b[p]], rows_b[p], sem_vb[p])
        cp_w = pltpu.async_copy(w_hbm.at[idx_b[p]], w_b[p], sem_wb[p])
        return cp_v, cp_w

    pend = start_chunk(0)
    for c in range(NCHUNK):
        p = c & 1
        rows_v = rows_b[p]
        w_v = w_b[p]
        pend[0].wait()
        pend[1].wait()
        if c + 1 < NCHUNK:
            pend = start_chunk(c + 1)

        def group_body(g, carry2, rows_v=rows_v, w_v=w_v):
            rowbase = (g * 16 + lanes) * F

            lin0 = plsc.load_gather(w_v, [rowbase])
            lin1 = plsc.load_gather(w_v, [rowbase + 1])
            for f in range(2, F, 2):
                lin0 = lin0 + plsc.load_gather(w_v, [rowbase + f])
                lin1 = lin1 + plsc.load_gather(w_v, [rowbase + f + 1])
            lin = lin0 + lin1

            acc_sq = jnp.zeros((16,), jnp.float32)
            acc_q0 = jnp.zeros((16,), jnp.float32)
            acc_q1 = jnp.zeros((16,), jnp.float32)
            for k in range(K):
                kvec = jnp.full((16,), k, jnp.int32)
                v0 = plsc.load_gather(rows_v, [rowbase, kvec])
                v1 = plsc.load_gather(rows_v, [rowbase + 1, kvec])
                s0 = v0
                s1 = v1
                q0 = v0 * v0
                q1 = v1 * v1
                for f in range(2, F, 2):
                    a0 = plsc.load_gather(rows_v, [rowbase + f, kvec])
                    a1 = plsc.load_gather(rows_v, [rowbase + f + 1, kvec])
                    s0 = s0 + a0
                    q0 = q0 + a0 * a0
                    s1 = s1 + a1
                    q1 = q1 + a1 * a1
                s_k = s0 + s1
                acc_sq = acc_sq + s_k * s_k
                acc_q0 = acc_q0 + q0
                acc_q1 = acc_q1 + q1
            inter = 0.5 * (acc_sq - (acc_q0 + acc_q1))

            z = lin + inter
            out = 1.0 / (1.0 + jnp.exp(-z))
            lin_v[pl.ds(g * 16, 16)] = lin
            int_v[pl.ds(g * 16, 16)] = inter
            o_v[pl.ds(g * 16, 16)] = out
            return carry2

        lax.fori_loop(0, NG, group_body, 0)

        sbase = wid * SPW + c * CHUNK
        pltpu.sync_copy(lin_v, lin_hbm.at[pl.ds(sbase, CHUNK)])
        pltpu.sync_copy(int_v, int_hbm.at[pl.ds(sbase, CHUNK)])
        pltpu.sync_copy(o_v, out_hbm.at[pl.ds(sbase, CHUNK)])


_fm_kernel = pl.kernel(
    _fm_body,
    out_type=(
        jax.ShapeDtypeStruct((B,), jnp.float32),
        jax.ShapeDtypeStruct((B,), jnp.float32),
        jax.ShapeDtypeStruct((B,), jnp.float32),
    ),
    mesh=plsc.VectorSubcoreMesh(core_axis_name="c", subcore_axis_name="s"),
    compiler_params=pltpu.CompilerParams(
        needs_layout_passes=False, use_tc_tiling_on_sc=False),
    scratch_types=(
        pltpu.VMEM((ROWS,), jnp.int32),
        pltpu.VMEM((ROWS,), jnp.int32),
        pltpu.VMEM((ROWS, K), jnp.float32),
        pltpu.VMEM((ROWS, K), jnp.float32),
        pltpu.VMEM((ROWS,), jnp.float32),
        pltpu.VMEM((ROWS,), jnp.float32),
        pltpu.VMEM((CHUNK,), jnp.float32),
        pltpu.VMEM((CHUNK,), jnp.float32),
        pltpu.VMEM((CHUNK,), jnp.float32),
        pltpu.SemaphoreType.DMA,
        pltpu.SemaphoreType.DMA,
        pltpu.SemaphoreType.DMA,
        pltpu.SemaphoreType.DMA,
    ),
)


@jax.jit
def kernel(x, w_table, V_table):
    xf = x.reshape(-1)
    out, lin, inter = _fm_kernel(xf, w_table.reshape(-1), V_table)
    return (out.reshape(B, 1), lin.reshape(B, 1), inter.reshape(B, 1))


# speedup vs baseline: 1.2145x; 1.0035x over previous
import jax
import jax.numpy as jnp
from jax import lax
from jax.experimental import pallas as pl
from jax.experimental.pallas import tpu as pltpu
from jax.experimental.pallas import tpu_sc as plsc

B = 16384
F = 26
K = 16
NC = 2
NS = 16
NW = NC * NS
SPW = B // NW
CHUNK = 128
NCHUNK = SPW // CHUNK
ROWS = CHUNK * F
NG = CHUNK // 16


def _fm_body(xf_hbm, w_hbm, v_hbm, out_hbm, lin_hbm, int_hbm,
             idx_v0, idx_v1, rows_v0, rows_v1, w_v0, w_v1,
             lin_v, int_v, o_v, sem_v0, sem_v1, sem_w0, sem_w1):
    wid = lax.axis_index("s") * NC + lax.axis_index("c")
    lanes = lax.iota(jnp.int32, 16)
    idx_b = (idx_v0, idx_v1)
    rows_b = (rows_v0, rows_v1)
    w_b = (w_v0, w_v1)
    sem_vb = (sem_v0, sem_v1)
    sem_wb = (sem_w0, sem_w1)

    def start_chunk(c):
        p = c & 1
        sbase = wid * SPW + c * CHUNK
        pltpu.sync_copy(xf_hbm.at[pl.ds(sbase * F, ROWS)], idx_b[p])
        cp_v = pltpu.async_copy(v_hbm.at[idx_b[p]], rows_b[p], sem_vb[p])
        cp_w = pltpu.async_copy(w_hbm.at[idx_b[p]], w_b[p], sem_wb[p])
        return cp_v, cp_w

    pend = start_chunk(0)
    for c in range(NCHUNK):
        p = c & 1
        rows_v = rows_b[p]
        w_v = w_b[p]
        pend[0].wait()
        pend[1].wait()
        if c + 1 < NCHUNK:
            pend = start_chunk(c + 1)

        def group_body(g, carry2, rows_v=rows_v, w_v=w_v):
            rowbase = (g * 16 + lanes) * F

            lin0 = plsc.load_gather(w_v, [rowbase])
            lin1 = plsc.load_gather(w_v, [rowbase + 1])
            for f in range(2, F, 2):
                lin0 = lin0 + plsc.load_gather(w_v, [rowbase + f])
                lin1 = lin1 + plsc.load_gather(w_v, [rowbase + f + 1])
            lin = lin0 + lin1

            acc_sq = jnp.zeros((16,), jnp.float32)
            acc_q0 = jnp.zeros((16,), jnp.float32)
            acc_q1 = jnp.zeros((16,), jnp.float32)
            for k in range(K):
                kvec = jnp.full((16,), k, jnp.int32)
                v0 = plsc.load_gather(rows_v, [rowbase, kvec])
                v1 = plsc.load_gather(rows_v, [rowbase + 1, kvec])
                s0 = v0
                s1 = v1
                q0 = v0 * v0
                q1 = v1 * v1
                for f in range(2, F, 2):
                    a0 = plsc.load_gather(rows_v, [rowbase + f, kvec])
                    a1 = plsc.load_gather(rows_v, [rowbase + f + 1, kvec])
                    s0 = s0 + a0
                    q0 = q0 + a0 * a0
                    s1 = s1 + a1
                    q1 = q1 + a1 * a1
                s_k = s0 + s1
                acc_sq = acc_sq + s_k * s_k
                acc_q0 = acc_q0 + q0
                acc_q1 = acc_q1 + q1
            inter = 0.5 * (acc_sq - (acc_q0 + acc_q1))

            z = lin + inter
            out = 1.0 / (1.0 + jnp.exp(-z))
            lin_v[pl.ds(g * 16, 16)] = lin
            int_v[pl.ds(g * 16, 16)] = inter
            o_v[pl.ds(g * 16, 16)] = out
            return carry2

        lax.fori_loop(0, NG, group_body, 0)

        sbase = wid * SPW + c * CHUNK
        pltpu.sync_copy(lin_v, lin_hbm.at[pl.ds(sbase, CHUNK)])
        pltpu.sync_copy(int_v, int_hbm.at[pl.ds(sbase, CHUNK)])
        pltpu.sync_copy(o_v, out_hbm.at[pl.ds(sbase, CHUNK)])


_fm_kernel = pl.kernel(
    _fm_body,
    out_type=(
        jax.ShapeDtypeStruct((B,), jnp.float32),
        jax.ShapeDtypeStruct((B,), jnp.float32),
        jax.ShapeDtypeStruct((B,), jnp.float32),
    ),
    mesh=plsc.VectorSubcoreMesh(core_axis_name="c", subcore_axis_name="s"),
    compiler_params=pltpu.CompilerParams(
        needs_layout_passes=False, use_tc_tiling_on_sc=False),
    scratch_types=(
        pltpu.VMEM((ROWS,), jnp.int32),
        pltpu.VMEM((ROWS,), jnp.int32),
        pltpu.VMEM((ROWS, K), jnp.float32),
        pltpu.VMEM((ROWS, K), jnp.float32),
        pltpu.VMEM((ROWS,), jnp.float32),
        pltpu.VMEM((ROWS,), jnp.float32),
        pltpu.VMEM((CHUNK,), jnp.float32),
        pltpu.VMEM((CHUNK,), jnp.float32),
        pltpu.VMEM((CHUNK,), jnp.float32),
        pltpu.SemaphoreType.DMA,
        pltpu.SemaphoreType.DMA,
        pltpu.SemaphoreType.DMA,
        pltpu.SemaphoreType.DMA,
    ),
)


@jax.jit
def kernel(x, w_table, V_table):
    xf = x.reshape(-1)
    out, lin, inter = _fm_kernel(xf, w_table.reshape(-1), V_table)
    return (out.reshape(B, 1), lin.reshape(B, 1), inter.reshape(B, 1))
